# baseline (speedup 1.0000x reference)
import functools

import jax
import jax.numpy as jnp
from jax import lax
from jax.experimental import pallas as pl
from jax.experimental.pallas import tpu as pltpu
from jax.experimental.pallas import tpu_sc as plsc

_B, _T, _D, _K = 8, 1024, 32, 8192
_N = _B * _T
_TILE_T = 1024
_GRID = _N // _TILE_T

_NC, _NS = 2, 16
_NW = _NC * _NS
_BPW = _N // _NW
_CH = 128
_NCH = _BPW // _CH


def _tc_body(z_ref, zn_ref, cb_ref, usage_ref, idx_ref, loss_ref, ent_ref,
             ppl_ref, cbt_ref, en_ref):
    i = pl.program_id(0)
    zb = z_ref[...]
    zn = zn_ref[...]

    @pl.when(i == 0)
    def _prep():
        cbt0 = jnp.transpose(cb_ref[...], (1, 0))
        cbt_ref[...] = cbt0
        en_ref[...] = jnp.sum(cbt0 * cbt0, axis=0, keepdims=True)

    cbt = cbt_ref[...]
    en = en_ref[...]
    dot = jnp.dot(zb, cbt, preferred_element_type=jnp.float32)
    dist = zn + en - 2.0 * dot
    md = jnp.min(dist, axis=-1)
    idx = jnp.argmin(dist, axis=-1).astype(jnp.int32)
    idx_ref[...] = idx.reshape(_TILE_T // 128, 128)

    @pl.when(i == 0)
    def _init():
        loss_ref[0, 0] = 0.0

    loss_ref[0, 0] += jnp.sum(md)

    @pl.when(i == _GRID - 1)
    def _finish():
        loss_ref[0, 0] = loss_ref[0, 0] / float(_N * _D)
        u = usage_ref[...]
        s = jnp.sum(u)
        p = u / (s + 1e-10)
        ent = -jnp.sum(p * jnp.log(p + 1e-10))
        ent_ref[0, 0] = ent
        ppl_ref[0, 0] = jnp.exp(ent)


def _tc_call(z_flat, zn, codebook, usage2d):
    return pl.pallas_call(
        _tc_body,
        grid=(_GRID,),
        in_specs=[
            pl.BlockSpec((_TILE_T, _D), lambda i: (i, 0)),
            pl.BlockSpec((_TILE_T, 1), lambda i: (i, 0)),
            pl.BlockSpec((_K, _D), lambda i: (0, 0)),
            pl.BlockSpec(usage2d.shape, lambda i: (0, 0)),
        ],
        scratch_shapes=[
            pltpu.VMEM((_D, _K), jnp.float32),
            pltpu.VMEM((1, _K), jnp.float32),
        ],
        out_specs=[
            pl.BlockSpec((_TILE_T // 128, 128), lambda i: (i, 0)),
            pl.BlockSpec((1, 1), lambda i: (0, 0), memory_space=pltpu.SMEM),
            pl.BlockSpec((1, 1), lambda i: (0, 0), memory_space=pltpu.SMEM),
            pl.BlockSpec((1, 1), lambda i: (0, 0), memory_space=pltpu.SMEM),
        ],
        out_shape=[
            jax.ShapeDtypeStruct((_N // 128, 128), jnp.int32),
            jax.ShapeDtypeStruct((1, 1), jnp.float32),
            jax.ShapeDtypeStruct((1, 1), jnp.float32),
            jax.ShapeDtypeStruct((1, 1), jnp.float32),
        ],
    )(z_flat, zn, codebook, usage2d)


def _sc_gather(codebook, idx2d):
    mesh = plsc.VectorSubcoreMesh(core_axis_name="c", subcore_axis_name="s")

    @functools.partial(
        pl.kernel,
        mesh=mesh,
        compiler_params=pltpu.CompilerParams(use_tc_tiling_on_sc=False),
        out_type=jax.ShapeDtypeStruct((_N, _D), jnp.float32),
        scratch_types=[
            pltpu.VMEM((_NCH, _CH), jnp.int32),
            pltpu.VMEM((_BPW, _D), jnp.float32),
            pltpu.SemaphoreType.DMA,
        ],
    )
    def gather_kernel(cb_hbm, idx_hbm, out_hbm, idx_v, rows_v, sem):
        wid = lax.axis_index("s") * _NC + lax.axis_index("c")
        pltpu.sync_copy(idx_hbm.at[pl.ds(wid * _NCH, _NCH)], idx_v)
        copies = [
            pltpu.async_copy(cb_hbm.at[idx_v.at[j]],
                             rows_v.at[pl.ds(j * _CH, _CH)], sem)
            for j in range(_NCH)
        ]
        for cp in copies:
            cp.wait()
        pltpu.sync_copy(rows_v, out_hbm.at[pl.ds(wid * _BPW, _BPW)])

    return gather_kernel(codebook, idx2d)


def kernel(z, codebook, codebook_usage):
    z_flat = z.reshape(_N, _D)
    zn = jnp.sum(z_flat ** 2, axis=-1, keepdims=True)
    usage2d = codebook_usage.reshape(64, 128)
    idx2d, loss, ent, ppl = _tc_call(z_flat, zn, codebook, usage2d)
    z_q = _sc_gather(codebook, idx2d)
    indices = idx2d.reshape(_B, _T)
    return (
        z_q.reshape(_B, _T, _D),
        indices,
        loss.reshape(()),
        ppl.reshape(()),
        ent.reshape(()),
    )


# speedup vs baseline: 1.0181x; 1.0181x over previous
import functools

import jax
import jax.numpy as jnp
from jax import lax
from jax.experimental import pallas as pl
from jax.experimental.pallas import tpu as pltpu
from jax.experimental.pallas import tpu_sc as plsc

_B, _T, _D, _K = 8, 1024, 32, 8192
_N = _B * _T
_TILE_T = 1024
_GRID = _N // _TILE_T

_NC, _NS = 2, 16
_NW = _NC * _NS
_BPW = _N // _NW
_CH = 128
_NCH = _BPW // _CH


def _tc_body(z_ref, zn_ref, cbt_ref, usage_ref, idx_ref, loss_ref, ent_ref,
             ppl_ref):
    i = pl.program_id(0)
    zb = z_ref[...]
    zn = zn_ref[...]
    cbt = cbt_ref[...]
    en = jnp.sum(cbt * cbt, axis=0, keepdims=True)
    dot = jnp.dot(zb, cbt, preferred_element_type=jnp.float32)
    dist = zn + en - 2.0 * dot
    md = jnp.min(dist, axis=-1)
    idx = jnp.argmin(dist, axis=-1).astype(jnp.int32)
    idx_ref[...] = idx.reshape(_TILE_T // 128, 128)

    @pl.when(i == 0)
    def _init():
        loss_ref[0, 0] = 0.0

    loss_ref[0, 0] += jnp.sum(md)

    @pl.when(i == _GRID - 1)
    def _finish():
        loss_ref[0, 0] = loss_ref[0, 0] / float(_N * _D)
        u = usage_ref[...]
        s = jnp.sum(u)
        p = u / (s + 1e-10)
        ent = -jnp.sum(p * jnp.log(p + 1e-10))
        ent_ref[0, 0] = ent
        ppl_ref[0, 0] = jnp.exp(ent)


def _tc_call(z_flat, zn, cbt, usage2d):
    return pl.pallas_call(
        _tc_body,
        grid=(_GRID,),
        in_specs=[
            pl.BlockSpec((_TILE_T, _D), lambda i: (i, 0)),
            pl.BlockSpec((_TILE_T, 1), lambda i: (i, 0)),
            pl.BlockSpec((_D, _K), lambda i: (0, 0)),
            pl.BlockSpec(usage2d.shape, lambda i: (0, 0)),
        ],
        out_specs=[
            pl.BlockSpec((_TILE_T // 128, 128), lambda i: (i, 0)),
            pl.BlockSpec((1, 1), lambda i: (0, 0), memory_space=pltpu.SMEM),
            pl.BlockSpec((1, 1), lambda i: (0, 0), memory_space=pltpu.SMEM),
            pl.BlockSpec((1, 1), lambda i: (0, 0), memory_space=pltpu.SMEM),
        ],
        out_shape=[
            jax.ShapeDtypeStruct((_N // 128, 128), jnp.int32),
            jax.ShapeDtypeStruct((1, 1), jnp.float32),
            jax.ShapeDtypeStruct((1, 1), jnp.float32),
            jax.ShapeDtypeStruct((1, 1), jnp.float32),
        ],
    )(z_flat, zn, cbt, usage2d)


def _sc_gather(codebook, idx2d):
    mesh = plsc.VectorSubcoreMesh(core_axis_name="c", subcore_axis_name="s")

    @functools.partial(
        pl.kernel,
        mesh=mesh,
        compiler_params=pltpu.CompilerParams(use_tc_tiling_on_sc=False),
        out_type=jax.ShapeDtypeStruct((_N, _D), jnp.float32),
        scratch_types=[
            pltpu.VMEM((_NCH, _CH), jnp.int32),
            pltpu.VMEM((_BPW, _D), jnp.float32),
            pltpu.SemaphoreType.DMA,
        ],
    )
    def gather_kernel(cb_hbm, idx_hbm, out_hbm, idx_v, rows_v, sem):
        wid = lax.axis_index("s") * _NC + lax.axis_index("c")
        pltpu.sync_copy(idx_hbm.at[pl.ds(wid * _NCH, _NCH)], idx_v)
        copies = [
            pltpu.async_copy(cb_hbm.at[idx_v.at[j]],
                             rows_v.at[pl.ds(j * _CH, _CH)], sem)
            for j in range(_NCH)
        ]
        for cp in copies:
            cp.wait()
        pltpu.sync_copy(rows_v, out_hbm.at[pl.ds(wid * _BPW, _BPW)])

    return gather_kernel(codebook, idx2d)


def kernel(z, codebook, codebook_usage):
    z_flat = z.reshape(_N, _D)
    zn = jnp.sum(z_flat ** 2, axis=-1, keepdims=True)
    cbt = codebook.T
    usage2d = codebook_usage.reshape(64, 128)
    idx2d, loss, ent, ppl = _tc_call(z_flat, zn, cbt, usage2d)
    z_q = _sc_gather(codebook, idx2d)
    indices = idx2d.reshape(_B, _T)
    return (
        z_q.reshape(_B, _T, _D),
        indices,
        loss.reshape(()),
        ppl.reshape(()),
        ent.reshape(()),
    )


# speedup vs baseline: 1.0680x; 1.0491x over previous
import functools

import jax
import jax.numpy as jnp
from jax import lax
from jax.experimental import pallas as pl
from jax.experimental.pallas import tpu as pltpu
from jax.experimental.pallas import tpu_sc as plsc

_B, _T, _D, _K = 8, 1024, 32, 8192
_N = _B * _T
_TILE_T = 1024
_GRID = _N // _TILE_T

_NC, _NS = 2, 16
_NW = _NC * _NS
_BPW = _N // _NW
_CH = 128
_NCH = _BPW // _CH


def _tc_body(z_ref, zn_ref, cbt_ref, usage_ref, idx_ref, loss_ref, ent_ref,
             ppl_ref):
    i = pl.program_id(0)
    zb = z_ref[...]
    zn = zn_ref[...]
    cbt = cbt_ref[...]
    en = jnp.sum(cbt * cbt, axis=0, keepdims=True)
    dot = jnp.dot(zb * -2.0, cbt, preferred_element_type=jnp.float32)
    dist = zn + en + dot
    md = jnp.min(dist, axis=-1)
    idx = jnp.argmin(dist, axis=-1).astype(jnp.int32)
    idx_ref[...] = idx.reshape(_TILE_T // 128, 128)

    @pl.when(i == 0)
    def _init():
        loss_ref[0, 0] = 0.0

    loss_ref[0, 0] += jnp.sum(md)

    @pl.when(i == _GRID - 1)
    def _finish():
        loss_ref[0, 0] = loss_ref[0, 0] / float(_N * _D)
        u = usage_ref[...]
        s = jnp.sum(u)
        p = u / (s + 1e-10)
        ent = -jnp.sum(p * jnp.log(p + 1e-10))
        ent_ref[0, 0] = ent
        ppl_ref[0, 0] = jnp.exp(ent)


def _tc_call(z_flat, zn, cbt, usage2d):
    return pl.pallas_call(
        _tc_body,
        grid=(_GRID,),
        in_specs=[
            pl.BlockSpec((_TILE_T, _D), lambda i: (i, 0)),
            pl.BlockSpec((_TILE_T, 1), lambda i: (i, 0)),
            pl.BlockSpec((_D, _K), lambda i: (0, 0)),
            pl.BlockSpec(usage2d.shape, lambda i: (0, 0)),
        ],
        out_specs=[
            pl.BlockSpec((_TILE_T // 128, 128), lambda i: (i, 0)),
            pl.BlockSpec((1, 1), lambda i: (0, 0), memory_space=pltpu.SMEM),
            pl.BlockSpec((1, 1), lambda i: (0, 0), memory_space=pltpu.SMEM),
            pl.BlockSpec((1, 1), lambda i: (0, 0), memory_space=pltpu.SMEM),
        ],
        out_shape=[
            jax.ShapeDtypeStruct((_N // 128, 128), jnp.int32),
            jax.ShapeDtypeStruct((1, 1), jnp.float32),
            jax.ShapeDtypeStruct((1, 1), jnp.float32),
            jax.ShapeDtypeStruct((1, 1), jnp.float32),
        ],
    )(z_flat, zn, cbt, usage2d)


def _sc_gather(codebook, idx2d):
    mesh = plsc.VectorSubcoreMesh(core_axis_name="c", subcore_axis_name="s")

    @functools.partial(
        pl.kernel,
        mesh=mesh,
        compiler_params=pltpu.CompilerParams(use_tc_tiling_on_sc=False),
        out_type=jax.ShapeDtypeStruct((_N, _D), jnp.float32),
        scratch_types=[
            pltpu.VMEM((_NCH, _CH), jnp.int32),
            pltpu.VMEM((_BPW, _D), jnp.float32),
            pltpu.SemaphoreType.DMA,
        ],
    )
    def gather_kernel(cb_hbm, idx_hbm, out_hbm, idx_v, rows_v, sem):
        wid = lax.axis_index("s") * _NC + lax.axis_index("c")
        pltpu.sync_copy(idx_hbm.at[pl.ds(wid * _NCH, _NCH)], idx_v)
        copies = [
            pltpu.async_copy(cb_hbm.at[idx_v.at[j]],
                             rows_v.at[pl.ds(j * _CH, _CH)], sem)
            for j in range(_NCH)
        ]
        for cp in copies:
            cp.wait()
        pltpu.sync_copy(rows_v, out_hbm.at[pl.ds(wid * _BPW, _BPW)])

    return gather_kernel(codebook, idx2d)


def kernel(z, codebook, codebook_usage):
    z_flat = z.reshape(_N, _D)
    zn = jnp.sum(z_flat ** 2, axis=-1, keepdims=True)
    cbt = codebook.T
    usage2d = codebook_usage.reshape(64, 128)
    idx2d, loss, ent, ppl = _tc_call(z_flat, zn, cbt, usage2d)
    z_q = _sc_gather(codebook, idx2d)
    indices = idx2d.reshape(_B, _T)
    return (
        z_q.reshape(_B, _T, _D),
        indices,
        loss.reshape(()),
        ppl.reshape(()),
        ent.reshape(()),
    )
